# Initial kernel scaffold; baseline (speedup 1.0000x reference)
import jax
import jax.numpy as jnp
from jax.experimental import pallas as pl


def kernel(x, edge_index, edge_attr, batch, W_out_w, W_out_b, W_in_w, W_in_b, lin1_w, lin1_b, lin2_w, lin2_b):
    raise NotImplementedError("write your pallas kernel here")


# speedup vs baseline: 5.0343x; 5.0343x over previous
import functools

import jax
import jax.numpy as jnp
from jax import lax
from jax.experimental import pallas as pl
from jax.experimental.pallas import tpu as pltpu
from jax.experimental.pallas import tpu_sc as plsc

N = 10000
E = 320000
D = 128
DE = 16
NG = 256
INNER = 512

NC = 2
NS = 16
CH = 128
WCHUNKS = 80
EP = NC * NS * WCHUNKS * CH
TRASH = N
NACC = 10240
ZROWS = 640

BN = 1000
NB = N // BN


def _make_sc_scatter(F):
    mesh = plsc.VectorSubcoreMesh(core_axis_name="c", subcore_axis_name="s")

    @functools.partial(
        pl.kernel,
        out_type=jax.ShapeDtypeStruct((NC, NACC, F), jnp.float32),
        mesh=mesh,
        scratch_types=[
            pltpu.VMEM((WCHUNKS, CH), jnp.int32),
            pltpu.VMEM((WCHUNKS, CH), jnp.int32),
            pltpu.VMEM((CH, F), jnp.float32),
            pltpu.VMEM_SHARED((NACC, F), jnp.float32),
            pltpu.SemaphoreType.DMA,
        ],
    )
    def sc_scatter(vals_hbm, src_hbm, dst_hbm, zeros_hbm, out_hbm,
                   src_v, dst_v, rows_v, acc_sh, sem):
        cid = lax.axis_index("c")
        sid = lax.axis_index("s")
        gid = cid * NS + sid
        pltpu.sync_copy(zeros_hbm, acc_sh.at[pl.ds(sid * ZROWS, ZROWS)])
        pltpu.sync_copy(src_hbm.at[pl.ds(gid * WCHUNKS, WCHUNKS)], src_v)
        pltpu.sync_copy(dst_hbm.at[pl.ds(gid * WCHUNKS, WCHUNKS)], dst_v)
        plsc.subcore_barrier()

        @pl.loop(0, WCHUNKS)
        def _(ci):
            pltpu.async_copy(vals_hbm.at[src_v.at[ci]], rows_v, sem).wait()
            pltpu.sync_copy(rows_v, acc_sh.at[dst_v.at[ci]], add=True)

        plsc.subcore_barrier()
        pltpu.sync_copy(acc_sh.at[pl.ds(sid * ZROWS, ZROWS)],
                        out_hbm.at[cid, pl.ds(sid * ZROWS, ZROWS)])

    return sc_scatter


def _make_sc_scatter_linear():
    mesh = plsc.VectorSubcoreMesh(core_axis_name="c", subcore_axis_name="s")

    @functools.partial(
        pl.kernel,
        out_type=jax.ShapeDtypeStruct((NC, NACC, D), jnp.float32),
        mesh=mesh,
        scratch_types=[
            pltpu.VMEM((WCHUNKS, CH), jnp.int32),
            pltpu.VMEM((CH, D), jnp.float32),
            pltpu.VMEM_SHARED((NACC, D), jnp.float32),
        ],
    )
    def sc_scatter_lin(vals_hbm, dst_hbm, zeros_hbm, out_hbm,
                       dst_v, rows_v, acc_sh):
        cid = lax.axis_index("c")
        sid = lax.axis_index("s")
        gid = cid * NS + sid
        pltpu.sync_copy(zeros_hbm, acc_sh.at[pl.ds(sid * ZROWS, ZROWS)])
        pltpu.sync_copy(dst_hbm.at[pl.ds(gid * WCHUNKS, WCHUNKS)], dst_v)
        plsc.subcore_barrier()

        @pl.loop(0, WCHUNKS)
        def _(ci):
            chunk = gid * WCHUNKS + ci

            @pl.when(chunk < E // CH)
            def _():
                pltpu.sync_copy(vals_hbm.at[pl.ds(chunk * CH, CH)], rows_v)
                pltpu.sync_copy(rows_v, acc_sh.at[dst_v.at[ci]], add=True)

        plsc.subcore_barrier()
        pltpu.sync_copy(acc_sh.at[pl.ds(sid * ZROWS, ZROWS)],
                        out_hbm.at[cid, pl.ds(sid * ZROWS, ZROWS)])

    return sc_scatter_lin


_sc_scatter_feat = _make_sc_scatter(D)
_sc_scatter_edge = _make_sc_scatter_linear()


def _softmax_pool_y(xb, woutT, woutb, w1t, batch_blk, pin_ref, pout_ref, y_ref):
    z = jnp.dot(xb, woutT, preferred_element_type=jnp.float32) + woutb
    z = z - jnp.max(z, axis=1, keepdims=True)
    ez = jnp.exp(z)
    fp = ez / jnp.sum(ez, axis=1, keepdims=True)
    gids = lax.broadcasted_iota(jnp.int32, (NG, BN), 0)
    oh = jnp.where(batch_blk[None, :] == gids, 1.0, 0.0)

    @pl.when(pl.program_id(0) == 0)
    def _():
        pout_ref[...] = pin_ref[...]

    pout_ref[...] += jnp.dot(oh, fp, preferred_element_type=jnp.float32)
    if y_ref is not None:
        y_ref[...] = jnp.dot(xb, w1t, preferred_element_type=jnp.float32)


def _iter_first_body(x_ref, woutT_ref, woutb_ref, w1t_ref, batch_ref, pin_ref,
                     pout_ref, y_ref):
    _softmax_pool_y(x_ref[...], woutT_ref[...], woutb_ref[...], w1t_ref[...],
                    batch_ref[0, 0, :], pin_ref, pout_ref, y_ref)


def _iter_rest_body(s_ref, yprev_ref, eagg_ref, winb_ref,
                    woutT_ref, woutb_ref, w1t_ref, batch_ref, pin_ref,
                    pout_ref, y_ref, *, last):
    xb = (s_ref[0] + s_ref[1] + yprev_ref[...]
          + eagg_ref[0] + eagg_ref[1]
          + winb_ref[...])
    _softmax_pool_y(xb, woutT_ref[...], woutb_ref[...],
                    None if last else w1t_ref[...],
                    batch_ref[0, 0, :], pin_ref, pout_ref,
                    None if last else y_ref)


def _ea_proj_body(ea_ref, w2t_ref, o_ref):
    o_ref[...] = jnp.dot(ea_ref[...], w2t_ref[...],
                         preferred_element_type=jnp.float32)


EB = 4000


def _ea_proj_call(edge_attr, w2t):
    return pl.pallas_call(
        _ea_proj_body,
        grid=(E // EB,),
        in_specs=[pl.BlockSpec((EB, DE), lambda i: (i, 0)),
                  _const_spec((DE, D))],
        out_specs=pl.BlockSpec((EB, D), lambda i: (i, 0)),
        out_shape=jax.ShapeDtypeStruct((E, D), jnp.float32),
    )(edge_attr, w2t)


def _mlp_body(p_ref, l1wT_ref, l1b_ref, l2wT_ref, l2b_ref, o_ref):
    h = (jnp.dot(p_ref[...], l1wT_ref[...], preferred_element_type=jnp.float32)
         + l1b_ref[...])
    o = (jnp.dot(h, l2wT_ref[...], preferred_element_type=jnp.float32)
         + l2b_ref[...])
    o_ref[...] = 1.0 / (1.0 + jnp.exp(-o))


def _row_spec(feat):
    return pl.BlockSpec((BN, feat), lambda i: (i, 0))


def _const_spec(shape):
    return pl.BlockSpec(shape, lambda i: tuple(0 for _ in shape))


_COMMON_SPECS = dict(
    woutT=_const_spec((D, INNER)),
    woutb=_const_spec((1, INNER)),
    w1t=_const_spec((D, D)),
    batch=pl.BlockSpec((1, 1, BN), lambda i: (i, 0, 0)),
    pin=_const_spec((NG, INNER)),
)


def _iter_first_call(x, woutT, woutb, w1t, batch3d, pooled):
    return pl.pallas_call(
        _iter_first_body,
        grid=(NB,),
        in_specs=[
            _row_spec(D), _COMMON_SPECS["woutT"], _COMMON_SPECS["woutb"],
            _COMMON_SPECS["w1t"], _COMMON_SPECS["batch"], _COMMON_SPECS["pin"],
        ],
        out_specs=[_const_spec((NG, INNER)), _row_spec(D)],
        out_shape=[
            jax.ShapeDtypeStruct((NG, INNER), jnp.float32),
            jax.ShapeDtypeStruct((N, D), jnp.float32),
        ],
    )(x, woutT, woutb, w1t, batch3d, pooled)


def _iter_rest_call(s2, yprev, eagg2, winb, woutT, woutb, w1t, batch3d,
                    pooled, last):
    out_specs = [_const_spec((NG, INNER))]
    out_shape = [jax.ShapeDtypeStruct((NG, INNER), jnp.float32)]
    if not last:
        out_specs.append(_row_spec(D))
        out_shape.append(jax.ShapeDtypeStruct((N, D), jnp.float32))
    body = functools.partial(_iter_rest_body, last=last)
    if last:
        def body(*refs):
            _iter_rest_body(*refs, None, last=True)
    return pl.pallas_call(
        body,
        grid=(NB,),
        in_specs=[
            pl.BlockSpec((NC, BN, D), lambda i: (0, i, 0)),
            _row_spec(D),
            pl.BlockSpec((NC, BN, D), lambda i: (0, i, 0)),
            _const_spec((1, D)),
            _COMMON_SPECS["woutT"], _COMMON_SPECS["woutb"],
            _COMMON_SPECS["w1t"], _COMMON_SPECS["batch"], _COMMON_SPECS["pin"],
        ],
        out_specs=out_specs,
        out_shape=out_shape,
    )(s2, yprev, eagg2, winb, woutT, woutb, w1t, batch3d, pooled)


def _mlp_call(pooled, l1wT, l1b, l2wT, l2b):
    return pl.pallas_call(
        _mlp_body,
        out_shape=jax.ShapeDtypeStruct((NG, 1), jnp.float32),
    )(pooled, l1wT, l1b, l2wT, l2b)


def kernel(x, edge_index, edge_attr, batch,
           W_out_w, W_out_b, W_in_w, W_in_b,
           lin1_w, lin1_b, lin2_w, lin2_b):
    f32 = jnp.float32
    woutT = W_out_w.T
    woutb = W_out_b.reshape(1, INNER)
    w1t = W_in_w[:, :D].T
    w2t = W_in_w[:, D:].T
    winb = W_in_b.reshape(1, D)

    pad = EP - E
    src = jnp.concatenate([edge_index[0], jnp.zeros((pad,), jnp.int32)])
    dst = jnp.concatenate([edge_index[1],
                           jnp.full((pad,), TRASH, jnp.int32)])
    srcp = src.reshape(EP // CH, CH)
    dstp = dst.reshape(EP // CH, CH)
    batch3d = batch.reshape(NB, 1, BN)
    zeros_f = jnp.zeros((ZROWS, D), f32)

    H = 64
    l1wT = jnp.zeros((INNER, H), f32).at[:, :50].set(lin1_w.T)
    l1b = jnp.zeros((1, H), f32).at[0, :50].set(lin1_b)
    l2wT = jnp.zeros((H, 1), f32).at[:50, 0].set(lin2_w[0])
    l2b = lin2_b.reshape(1, 1)

    ea128 = _ea_proj_call(edge_attr, w2t)
    eagg2 = _sc_scatter_edge(ea128, dstp, zeros_f)

    pooled = jnp.zeros((NG, INNER), f32)
    pooled, y = _iter_first_call(x, woutT, woutb, w1t, batch3d, pooled)
    for t in range(3):
        s2 = _sc_scatter_feat(y, srcp, dstp, zeros_f)
        res = _iter_rest_call(s2, y, eagg2, winb, woutT, woutb, w1t,
                              batch3d, pooled, last=(t == 2))
        if t == 2:
            (pooled,) = res
        else:
            pooled, y = res

    return _mlp_call(pooled, l1wT, l1b, l2wT, l2b)


# speedup vs baseline: 5.7798x; 1.1481x over previous
import functools

import jax
import jax.numpy as jnp
from jax import lax
from jax.experimental import pallas as pl
from jax.experimental.pallas import tpu as pltpu
from jax.experimental.pallas import tpu_sc as plsc

N = 10000
E = 320000
D = 128
DE = 16
NG = 256
INNER = 512

NC = 2
NS = 16
CH = 128
WCHUNKS = 80
EP = NC * NS * WCHUNKS * CH
TRASH = N
NACC = 10240
ZROWS = 640

BN = 1000
NB = N // BN


def _make_sc_scatter(F):
    mesh = plsc.VectorSubcoreMesh(core_axis_name="c", subcore_axis_name="s")

    @functools.partial(
        pl.kernel,
        out_type=jax.ShapeDtypeStruct((NC, NACC, F), jnp.float32),
        mesh=mesh,
        scratch_types=[
            pltpu.VMEM((WCHUNKS, CH), jnp.int32),
            pltpu.VMEM((CH,), jnp.int32),
            pltpu.VMEM((CH,), jnp.int32),
            pltpu.VMEM((CH,), jnp.int32),
            pltpu.VMEM((CH,), jnp.int32),
            pltpu.VMEM((CH, F), jnp.float32),
            pltpu.VMEM((CH, F), jnp.float32),
            pltpu.VMEM_SHARED((NACC, F), jnp.float32),
            pltpu.SemaphoreType.DMA,
            pltpu.SemaphoreType.DMA,
        ],
    )
    def sc_scatter(vals_hbm, packed_hbm, zeros_hbm, out_hbm,
                   packed_v, src_a, dst_a, src_b, dst_b, rows_a, rows_b,
                   acc_sh, sem_a, sem_b):
        cid = lax.axis_index("c")
        sid = lax.axis_index("s")
        gid = cid * NS + sid
        pltpu.sync_copy(zeros_hbm, acc_sh.at[pl.ds(sid * ZROWS, ZROWS)])
        pltpu.sync_copy(packed_hbm.at[pl.ds(gid * WCHUNKS, WCHUNKS)], packed_v)
        plsc.subcore_barrier()

        def unpack(ci, sbuf, dbuf):
            for j in range(CH // 16):
                sl = pl.ds(j * 16, 16)
                v = packed_v[ci, sl]
                sbuf[sl] = lax.bitwise_and(v, jnp.int32(0xFFFF))
                dbuf[sl] = lax.shift_right_logical(v, jnp.int32(16))

        unpack(0, src_a, dst_a)
        unpack(1, src_b, dst_b)
        pltpu.async_copy(vals_hbm.at[src_a], rows_a, sem_a)

        @pl.loop(0, WCHUNKS // 2)
        def _(cj):
            ci = 2 * cj
            pltpu.async_copy(vals_hbm.at[src_b], rows_b, sem_b)
            pltpu.make_async_copy(vals_hbm.at[src_a], rows_a, sem_a).wait()
            pltpu.sync_copy(rows_a, acc_sh.at[dst_a], add=True)

            @pl.when(ci + 2 < WCHUNKS)
            def _():
                unpack(ci + 2, src_a, dst_a)
                pltpu.async_copy(vals_hbm.at[src_a], rows_a, sem_a)

            pltpu.make_async_copy(vals_hbm.at[src_b], rows_b, sem_b).wait()
            pltpu.sync_copy(rows_b, acc_sh.at[dst_b], add=True)

            @pl.when(ci + 3 < WCHUNKS)
            def _():
                unpack(ci + 3, src_b, dst_b)

        plsc.subcore_barrier()
        pltpu.sync_copy(acc_sh.at[pl.ds(sid * ZROWS, ZROWS)],
                        out_hbm.at[cid, pl.ds(sid * ZROWS, ZROWS)])

    return sc_scatter


def _make_sc_scatter_linear():
    mesh = plsc.VectorSubcoreMesh(core_axis_name="c", subcore_axis_name="s")

    @functools.partial(
        pl.kernel,
        out_type=jax.ShapeDtypeStruct((NC, NACC, D), jnp.float32),
        mesh=mesh,
        scratch_types=[
            pltpu.VMEM((WCHUNKS, CH), jnp.int32),
            pltpu.VMEM((CH, D), jnp.float32),
            pltpu.VMEM_SHARED((NACC, D), jnp.float32),
        ],
    )
    def sc_scatter_lin(vals_hbm, dst_hbm, zeros_hbm, out_hbm,
                       dst_v, rows_v, acc_sh):
        cid = lax.axis_index("c")
        sid = lax.axis_index("s")
        gid = cid * NS + sid
        pltpu.sync_copy(zeros_hbm, acc_sh.at[pl.ds(sid * ZROWS, ZROWS)])
        pltpu.sync_copy(dst_hbm.at[pl.ds(gid * WCHUNKS, WCHUNKS)], dst_v)
        plsc.subcore_barrier()

        @pl.loop(0, WCHUNKS)
        def _(ci):
            chunk = gid * WCHUNKS + ci

            @pl.when(chunk < E // CH)
            def _():
                pltpu.sync_copy(vals_hbm.at[pl.ds(chunk * CH, CH)], rows_v)
                pltpu.sync_copy(rows_v, acc_sh.at[dst_v.at[ci]], add=True)

        plsc.subcore_barrier()
        pltpu.sync_copy(acc_sh.at[pl.ds(sid * ZROWS, ZROWS)],
                        out_hbm.at[cid, pl.ds(sid * ZROWS, ZROWS)])

    return sc_scatter_lin


_sc_scatter_feat = _make_sc_scatter(D)
_sc_scatter_edge = _make_sc_scatter_linear()


def _softmax_pool_y(xb, woutT, woutb, w1t, batch_blk, pin_ref, pout_ref, y_ref):
    z = jnp.dot(xb, woutT, preferred_element_type=jnp.float32) + woutb
    z = z - jnp.max(z, axis=1, keepdims=True)
    ez = jnp.exp(z)
    fp = ez / jnp.sum(ez, axis=1, keepdims=True)
    gids = lax.broadcasted_iota(jnp.int32, (NG, BN), 0)
    oh = jnp.where(batch_blk[None, :] == gids, 1.0, 0.0)

    @pl.when(pl.program_id(0) == 0)
    def _():
        pout_ref[...] = pin_ref[...]

    pout_ref[...] += jnp.dot(oh, fp, preferred_element_type=jnp.float32)
    if y_ref is not None:
        y_ref[...] = jnp.dot(xb, w1t, preferred_element_type=jnp.float32)


def _iter_first_body(x_ref, woutT_ref, woutb_ref, w1t_ref, batch_ref, pin_ref,
                     pout_ref, y_ref):
    _softmax_pool_y(x_ref[...], woutT_ref[...], woutb_ref[...], w1t_ref[...],
                    batch_ref[0, 0, :], pin_ref, pout_ref, y_ref)


def _iter_rest_body(s_ref, yprev_ref, eagg_ref, winb_ref,
                    woutT_ref, woutb_ref, w1t_ref, batch_ref, pin_ref,
                    pout_ref, y_ref, *, last):
    xb = (s_ref[0] + s_ref[1] + yprev_ref[...]
          + eagg_ref[0] + eagg_ref[1]
          + winb_ref[...])
    _softmax_pool_y(xb, woutT_ref[...], woutb_ref[...],
                    None if last else w1t_ref[...],
                    batch_ref[0, 0, :], pin_ref, pout_ref,
                    None if last else y_ref)


def _ea_proj_body(ea_ref, w2t_ref, o_ref):
    o_ref[...] = jnp.dot(ea_ref[...], w2t_ref[...],
                         preferred_element_type=jnp.float32)


EB = 4000


def _ea_proj_call(edge_attr, w2t):
    return pl.pallas_call(
        _ea_proj_body,
        grid=(E // EB,),
        in_specs=[pl.BlockSpec((EB, DE), lambda i: (i, 0)),
                  _const_spec((DE, D))],
        out_specs=pl.BlockSpec((EB, D), lambda i: (i, 0)),
        out_shape=jax.ShapeDtypeStruct((E, D), jnp.float32),
    )(edge_attr, w2t)


def _mlp_body(p_ref, l1wT_ref, l1b_ref, l2wT_ref, l2b_ref, o_ref):
    h = (jnp.dot(p_ref[...], l1wT_ref[...], preferred_element_type=jnp.float32)
         + l1b_ref[...])
    o = (jnp.dot(h, l2wT_ref[...], preferred_element_type=jnp.float32)
         + l2b_ref[...])
    o_ref[...] = 1.0 / (1.0 + jnp.exp(-o))


def _row_spec(feat):
    return pl.BlockSpec((BN, feat), lambda i: (i, 0))


def _const_spec(shape):
    return pl.BlockSpec(shape, lambda i: tuple(0 for _ in shape))


_COMMON_SPECS = dict(
    woutT=_const_spec((D, INNER)),
    woutb=_const_spec((1, INNER)),
    w1t=_const_spec((D, D)),
    batch=pl.BlockSpec((1, 1, BN), lambda i: (i, 0, 0)),
    pin=_const_spec((NG, INNER)),
)


def _iter_first_call(x, woutT, woutb, w1t, batch3d, pooled):
    return pl.pallas_call(
        _iter_first_body,
        grid=(NB,),
        in_specs=[
            _row_spec(D), _COMMON_SPECS["woutT"], _COMMON_SPECS["woutb"],
            _COMMON_SPECS["w1t"], _COMMON_SPECS["batch"], _COMMON_SPECS["pin"],
        ],
        out_specs=[_const_spec((NG, INNER)), _row_spec(D)],
        out_shape=[
            jax.ShapeDtypeStruct((NG, INNER), jnp.float32),
            jax.ShapeDtypeStruct((N, D), jnp.float32),
        ],
    )(x, woutT, woutb, w1t, batch3d, pooled)


def _iter_rest_call(s2, yprev, eagg2, winb, woutT, woutb, w1t, batch3d,
                    pooled, last):
    out_specs = [_const_spec((NG, INNER))]
    out_shape = [jax.ShapeDtypeStruct((NG, INNER), jnp.float32)]
    if not last:
        out_specs.append(_row_spec(D))
        out_shape.append(jax.ShapeDtypeStruct((N, D), jnp.float32))
    body = functools.partial(_iter_rest_body, last=last)
    if last:
        def body(*refs):
            _iter_rest_body(*refs, None, last=True)
    return pl.pallas_call(
        body,
        grid=(NB,),
        in_specs=[
            pl.BlockSpec((NC, BN, D), lambda i: (0, i, 0)),
            _row_spec(D),
            pl.BlockSpec((NC, BN, D), lambda i: (0, i, 0)),
            _const_spec((1, D)),
            _COMMON_SPECS["woutT"], _COMMON_SPECS["woutb"],
            _COMMON_SPECS["w1t"], _COMMON_SPECS["batch"], _COMMON_SPECS["pin"],
        ],
        out_specs=out_specs,
        out_shape=out_shape,
    )(s2, yprev, eagg2, winb, woutT, woutb, w1t, batch3d, pooled)


def _mlp_call(pooled, l1wT, l1b, l2wT, l2b):
    return pl.pallas_call(
        _mlp_body,
        out_shape=jax.ShapeDtypeStruct((NG, 1), jnp.float32),
    )(pooled, l1wT, l1b, l2wT, l2b)


def kernel(x, edge_index, edge_attr, batch,
           W_out_w, W_out_b, W_in_w, W_in_b,
           lin1_w, lin1_b, lin2_w, lin2_b):
    f32 = jnp.float32
    woutT = W_out_w.T
    woutb = W_out_b.reshape(1, INNER)
    w1t = W_in_w[:, :D].T
    w2t = W_in_w[:, D:].T
    winb = W_in_b.reshape(1, D)

    pad = EP - E
    src = jnp.concatenate([edge_index[0], jnp.zeros((pad,), jnp.int32)])
    dst = jnp.concatenate([edge_index[1],
                           jnp.full((pad,), TRASH, jnp.int32)])
    dstp = dst.reshape(EP // CH, CH)
    packedp = ((dst << 16) | src).reshape(EP // CH, CH)
    batch3d = batch.reshape(NB, 1, BN)
    zeros_f = jnp.zeros((ZROWS, D), f32)

    H = 64
    l1wT = jnp.zeros((INNER, H), f32).at[:, :50].set(lin1_w.T)
    l1b = jnp.zeros((1, H), f32).at[0, :50].set(lin1_b)
    l2wT = jnp.zeros((H, 1), f32).at[:50, 0].set(lin2_w[0])
    l2b = lin2_b.reshape(1, 1)

    ea128 = _ea_proj_call(edge_attr, w2t)
    eagg2 = _sc_scatter_edge(ea128, dstp, zeros_f)

    pooled = jnp.zeros((NG, INNER), f32)
    pooled, y = _iter_first_call(x, woutT, woutb, w1t, batch3d, pooled)
    for t in range(3):
        s2 = _sc_scatter_feat(y, packedp, zeros_f)
        res = _iter_rest_call(s2, y, eagg2, winb, woutT, woutb, w1t,
                              batch3d, pooled, last=(t == 2))
        if t == 2:
            (pooled,) = res
        else:
            pooled, y = res

    return _mlp_call(pooled, l1wT, l1b, l2wT, l2b)
